# Initial kernel scaffold; baseline (speedup 1.0000x reference)
import jax
import jax.numpy as jnp
from jax.experimental import pallas as pl


def kernel(confs, corrects):
    raise NotImplementedError("write your pallas kernel here")


# speedup vs baseline: 106.2857x; 106.2857x over previous
import functools

import jax
import jax.numpy as jnp
import numpy as np
from jax import lax
from jax.experimental import pallas as pl
from jax.experimental.pallas import tpu as pltpu, tpu_sc as plsc

N = 8388608
BINS = 20
NBANK = 32
LANES = 16
NWORKERS = 32
PER_TILE = N // NWORKERS
CHUNK = 32768
NCHUNK = PER_TILE // CHUNK


def _sc_partials(confs, corr_f):
    mesh = plsc.VectorSubcoreMesh(core_axis_name="c", subcore_axis_name="s")

    @functools.partial(
        pl.kernel,
        mesh=mesh,
        out_type=jax.ShapeDtypeStruct((3 * NWORKERS, 128), jnp.float32),
        compiler_params=pltpu.CompilerParams(needs_layout_passes=False),
        scratch_types=[
            pltpu.VMEM((CHUNK,), jnp.float32),
            pltpu.VMEM((CHUNK,), jnp.float32),
            pltpu.VMEM((LANES * NBANK,), jnp.float32),
            pltpu.VMEM((LANES * NBANK,), jnp.float32),
            pltpu.VMEM((LANES * NBANK,), jnp.float32),
            pltpu.VMEM((128,), jnp.float32),
        ],
    )
    def k(conf_hbm, corr_hbm, out_hbm, conf_v, corr_v, acc_c, acc_r, acc_n,
          row_v):
        wid = lax.axis_index("s") * 2 + lax.axis_index("c")
        zero16 = jnp.zeros((LANES,), jnp.float32)
        for i in range(NBANK):
            acc_c[pl.ds(i * LANES, LANES)] = zero16
            acc_r[pl.ds(i * LANES, LANES)] = zero16
            acc_n[pl.ds(i * LANES, LANES)] = zero16

        lane_off = lax.iota(jnp.int32, LANES) * NBANK
        ones = jnp.ones((LANES,), jnp.float32)
        base = wid * PER_TILE

        def chunk_body(c, _):
            off = base + c * CHUNK
            pltpu.sync_copy(conf_hbm.at[pl.ds(off, CHUNK)], conf_v)
            pltpu.sync_copy(corr_hbm.at[pl.ds(off, CHUNK)], corr_v)

            def vec_body(v, _):
                s = pl.ds(v * LANES, LANES)
                conf = conf_v[s]
                corr = corr_v[s]
                bi = jnp.minimum((conf * float(BINS)).astype(jnp.int32),
                                 BINS - 1)
                idx = bi + lane_off
                plsc.addupdate_scatter(acc_c, [idx], conf)
                plsc.addupdate_scatter(acc_r, [idx], corr)
                plsc.addupdate_scatter(acc_n, [idx], ones)
                return 0

            lax.fori_loop(0, CHUNK // LANES, vec_body, 0)
            return 0

        lax.fori_loop(0, NCHUNK, chunk_body, 0)

        for q, acc in enumerate((acc_c, acc_r, acc_n)):
            for i in range(128 // LANES):
                row_v[pl.ds(i * LANES, LANES)] = zero16
            lo = jnp.zeros((LANES,), jnp.float32)
            hi = jnp.zeros((LANES,), jnp.float32)
            for b in range(LANES):
                lo = lo + acc[pl.ds(b * NBANK, LANES)]
                hi = hi + acc[pl.ds(b * NBANK + LANES, LANES)]
            row_v[pl.ds(0, LANES)] = lo
            row_v[pl.ds(LANES, LANES)] = hi
            pltpu.sync_copy(row_v, out_hbm.at[q * NWORKERS + wid])

    return k(confs, corr_f)


def _finalize(partials):
    def fin(x_ref, o_ref):
        x = x_ref[...]
        conf_s = jnp.sum(x[0:32], axis=0, keepdims=True)
        corr_s = jnp.sum(x[32:64], axis=0, keepdims=True)
        cnt = jnp.sum(x[64:96], axis=0, keepdims=True)
        tiny = np.finfo(np.float32).tiny
        errs = jnp.abs(conf_s - corr_s) / (cnt + tiny)
        o_ref[...] = jnp.sum(errs * cnt / jnp.sum(cnt), keepdims=True)

    out = pl.pallas_call(
        fin,
        out_shape=jax.ShapeDtypeStruct((1, 1), jnp.float32),
    )(partials)
    return out[0, 0]


def kernel(confs, corrects):
    corr_f = corrects.astype(jnp.float32)
    partials = _sc_partials(confs, corr_f)
    return _finalize(partials)


# speedup vs baseline: 208.6578x; 1.9632x over previous
import functools

import jax
import jax.numpy as jnp
import numpy as np
from jax import lax
from jax.experimental import pallas as pl
from jax.experimental.pallas import tpu as pltpu, tpu_sc as plsc

N = 8388608
BINS = 20
NBANK = 32
LANES = 16
NWORKERS = 32
PER_TILE = N // NWORKERS
CHUNK = 32768
NCHUNK = PER_TILE // CHUNK


def _sc_partials(confs, corr_f):
    mesh = plsc.VectorSubcoreMesh(core_axis_name="c", subcore_axis_name="s")

    @functools.partial(
        pl.kernel,
        mesh=mesh,
        out_type=jax.ShapeDtypeStruct((3 * NWORKERS, 128), jnp.float32),
        compiler_params=pltpu.CompilerParams(needs_layout_passes=False),
        scratch_types=[
            pltpu.VMEM((CHUNK,), jnp.float32),
            pltpu.VMEM((CHUNK,), jnp.float32),
            pltpu.VMEM((LANES * NBANK,), jnp.float32),
            pltpu.VMEM((LANES * NBANK,), jnp.float32),
            pltpu.VMEM((LANES * NBANK,), jnp.float32),
            pltpu.VMEM((128,), jnp.float32),
        ],
    )
    def k(conf_hbm, corr_hbm, out_hbm, conf_v, corr_v, acc_c, acc_r, acc_n,
          row_v):
        wid = lax.axis_index("s") * 2 + lax.axis_index("c")
        zero16 = jnp.zeros((LANES,), jnp.float32)
        for i in range(NBANK):
            acc_c[pl.ds(i * LANES, LANES)] = zero16
            acc_r[pl.ds(i * LANES, LANES)] = zero16
            acc_n[pl.ds(i * LANES, LANES)] = zero16

        lane_off = lax.iota(jnp.int32, LANES) * NBANK
        ones = jnp.ones((LANES,), jnp.float32)
        base = wid * PER_TILE

        def chunk_body(c, _):
            off = base + c * CHUNK
            pltpu.sync_copy(conf_hbm.at[pl.ds(off, CHUNK)], conf_v)
            pltpu.sync_copy(corr_hbm.at[pl.ds(off, CHUNK)], corr_v)

            @plsc.parallel_loop(0, CHUNK // LANES, unroll=8)
            def vec_body(v):
                s = pl.ds(v * LANES, LANES)
                conf = conf_v[s]
                corr = corr_v[s]
                bi = jnp.minimum((conf * float(BINS)).astype(jnp.int32),
                                 BINS - 1)
                idx = bi + lane_off
                plsc.addupdate_scatter(acc_c, [idx], conf)
                plsc.addupdate_scatter(acc_r, [idx], corr)
                plsc.addupdate_scatter(acc_n, [idx], ones)

            return 0

        lax.fori_loop(0, NCHUNK, chunk_body, 0)

        for q, acc in enumerate((acc_c, acc_r, acc_n)):
            for i in range(128 // LANES):
                row_v[pl.ds(i * LANES, LANES)] = zero16
            lo = jnp.zeros((LANES,), jnp.float32)
            hi = jnp.zeros((LANES,), jnp.float32)
            for b in range(LANES):
                lo = lo + acc[pl.ds(b * NBANK, LANES)]
                hi = hi + acc[pl.ds(b * NBANK + LANES, LANES)]
            row_v[pl.ds(0, LANES)] = lo
            row_v[pl.ds(LANES, LANES)] = hi
            pltpu.sync_copy(row_v, out_hbm.at[q * NWORKERS + wid])

    return k(confs, corr_f)


def _finalize(partials):
    def fin(x_ref, o_ref):
        x = x_ref[...]
        conf_s = jnp.sum(x[0:32], axis=0, keepdims=True)
        corr_s = jnp.sum(x[32:64], axis=0, keepdims=True)
        cnt = jnp.sum(x[64:96], axis=0, keepdims=True)
        tiny = np.finfo(np.float32).tiny
        errs = jnp.abs(conf_s - corr_s) / (cnt + tiny)
        o_ref[...] = jnp.sum(errs * cnt / jnp.sum(cnt), keepdims=True)

    out = pl.pallas_call(
        fin,
        out_shape=jax.ShapeDtypeStruct((1, 1), jnp.float32),
    )(partials)
    return out[0, 0]


def kernel(confs, corrects):
    corr_f = corrects.astype(jnp.float32)
    partials = _sc_partials(confs, corr_f)
    return _finalize(partials)


# speedup vs baseline: 352.1111x; 1.6875x over previous
import functools

import jax
import jax.numpy as jnp
import numpy as np
from jax import lax
from jax.experimental import pallas as pl
from jax.experimental.pallas import tpu as pltpu, tpu_sc as plsc

N = 8388608
BINS = 20
NBANK = 32
LANES = 16
NWORKERS = 32
PER_TILE = N // NWORKERS
CHUNK = 16384
NCHUNK = PER_TILE // CHUNK


def _sc_partials(confs, corr_i):
    mesh = plsc.VectorSubcoreMesh(core_axis_name="c", subcore_axis_name="s")

    @functools.partial(
        pl.kernel,
        mesh=mesh,
        out_type=jax.ShapeDtypeStruct((3 * NWORKERS, 128), jnp.float32),
        compiler_params=pltpu.CompilerParams(needs_layout_passes=False),
        scratch_types=[
            pltpu.VMEM((CHUNK,), jnp.float32),
            pltpu.VMEM((CHUNK,), jnp.float32),
            pltpu.VMEM((CHUNK,), jnp.int32),
            pltpu.VMEM((CHUNK,), jnp.int32),
            pltpu.VMEM((LANES * NBANK,), jnp.float32),
            pltpu.VMEM((LANES * NBANK,), jnp.int32),
            pltpu.VMEM((128,), jnp.float32),
            pltpu.SemaphoreType.DMA,
            pltpu.SemaphoreType.DMA,
            pltpu.SemaphoreType.DMA,
            pltpu.SemaphoreType.DMA,
        ],
    )
    def k(conf_hbm, corr_hbm, out_hbm, conf_v0, conf_v1, corr_v0, corr_v1,
          acc_c, acc_p, row_v, semc0, semc1, semr0, semr1):
        wid = lax.axis_index("s") * 2 + lax.axis_index("c")
        zero16f = jnp.zeros((LANES,), jnp.float32)
        zero16i = jnp.zeros((LANES,), jnp.int32)
        for i in range(NBANK):
            acc_c[pl.ds(i * LANES, LANES)] = zero16f
            acc_p[pl.ds(i * LANES, LANES)] = zero16i

        lane_off = lax.iota(jnp.int32, LANES) * NBANK
        base = wid * PER_TILE
        conf_bufs = (conf_v0, conf_v1)
        corr_bufs = (corr_v0, corr_v1)
        semcs = (semc0, semc1)
        semrs = (semr0, semr1)

        def start(c):
            b = c % 2
            off = base + c * CHUNK
            dc = pltpu.async_copy(conf_hbm.at[pl.ds(off, CHUNK)],
                                  conf_bufs[b], semcs[b])
            dr = pltpu.async_copy(corr_hbm.at[pl.ds(off, CHUNK)],
                                  corr_bufs[b], semrs[b])
            return dc, dr

        pending = [None, None]
        pending[0] = start(0)
        for c in range(NCHUNK):
            if c + 1 < NCHUNK:
                pending[(c + 1) % 2] = start(c + 1)
            dc, dr = pending[c % 2]
            dc.wait()
            dr.wait()
            conf_v = conf_bufs[c % 2]
            corr_v = corr_bufs[c % 2]

            @plsc.parallel_loop(0, CHUNK // LANES, unroll=8)
            def vec_body(v):
                s = pl.ds(v * LANES, LANES)
                conf = conf_v[s]
                corr = corr_v[s]
                bi = jnp.minimum((conf * float(BINS)).astype(jnp.int32),
                                 BINS - 1)
                idx = bi + lane_off
                plsc.addupdate_scatter(acc_c, [idx], conf)
                plsc.addupdate_scatter(acc_p, [idx], (corr << 16) | 1)

        for i in range(128 // LANES):
            row_v[pl.ds(i * LANES, LANES)] = zero16f
        lo = jnp.zeros((LANES,), jnp.float32)
        hi = jnp.zeros((LANES,), jnp.float32)
        for b in range(LANES):
            lo = lo + acc_c[pl.ds(b * NBANK, LANES)]
            hi = hi + acc_c[pl.ds(b * NBANK + LANES, LANES)]
        row_v[pl.ds(0, LANES)] = lo
        row_v[pl.ds(LANES, LANES)] = hi
        pltpu.sync_copy(row_v, out_hbm.at[wid])

        cor_lo = jnp.zeros((LANES,), jnp.int32)
        cor_hi = jnp.zeros((LANES,), jnp.int32)
        cnt_lo = jnp.zeros((LANES,), jnp.int32)
        cnt_hi = jnp.zeros((LANES,), jnp.int32)
        for b in range(LANES):
            v_lo = acc_p[pl.ds(b * NBANK, LANES)]
            v_hi = acc_p[pl.ds(b * NBANK + LANES, LANES)]
            cor_lo = cor_lo + (v_lo >> 16)
            cor_hi = cor_hi + (v_hi >> 16)
            cnt_lo = cnt_lo + (v_lo & 0xFFFF)
            cnt_hi = cnt_hi + (v_hi & 0xFFFF)
        row_v[pl.ds(0, LANES)] = cor_lo.astype(jnp.float32)
        row_v[pl.ds(LANES, LANES)] = cor_hi.astype(jnp.float32)
        pltpu.sync_copy(row_v, out_hbm.at[NWORKERS + wid])
        row_v[pl.ds(0, LANES)] = cnt_lo.astype(jnp.float32)
        row_v[pl.ds(LANES, LANES)] = cnt_hi.astype(jnp.float32)
        pltpu.sync_copy(row_v, out_hbm.at[2 * NWORKERS + wid])

    return k(confs, corr_i)


def _finalize(partials):
    def fin(x_ref, o_ref):
        x = x_ref[...]
        conf_s = jnp.sum(x[0:32], axis=0, keepdims=True)
        corr_s = jnp.sum(x[32:64], axis=0, keepdims=True)
        cnt = jnp.sum(x[64:96], axis=0, keepdims=True)
        tiny = np.finfo(np.float32).tiny
        errs = jnp.abs(conf_s - corr_s) / (cnt + tiny)
        o_ref[...] = jnp.sum(errs * cnt / jnp.sum(cnt), keepdims=True)

    out = pl.pallas_call(
        fin,
        out_shape=jax.ShapeDtypeStruct((1, 1), jnp.float32),
    )(partials)
    return out[0, 0]


def kernel(confs, corrects):
    corr_i = corrects.astype(jnp.int32)
    partials = _sc_partials(confs, corr_i)
    return _finalize(partials)
